# baseline (speedup 1.0000x reference)
import functools

import jax
import jax.numpy as jnp
from jax import lax
from jax.experimental import pallas as pl
from jax.experimental.pallas import tpu as pltpu
from jax.experimental.pallas import tpu_sc as plsc

VOCAB = 100000
EMBED = 64
BATCH = 1024

_SC_INFO = plsc.get_sparse_core_info()
_NC = _SC_INFO.num_cores
_NS = _SC_INFO.num_subcores
_NW = _NC * _NS
_B_PER_W = BATCH // _NW


_NBUF = 8


def _sc_gather_body(idx_hbm, tab_hbm, out_hbm, idx_v, buf_v, rows_v, sem):
    wid = lax.axis_index("s") * _NC + lax.axis_index("c")
    base = wid * _B_PER_W
    pltpu.sync_copy(idx_hbm.at[pl.ds(base, _B_PER_W)], idx_v)
    lane_iota = lax.iota(jnp.int32, 16)
    scal = []
    for j in range(_B_PER_W):
        grp = idx_v[pl.ds((j // 16) * 16, 16)]
        scal.append(jnp.sum(jnp.where(lane_iota == (j % 16), grp, 0)))
    def _issue(j, s):
        blk = (scal[j] // 128) * 128
        pltpu.async_copy(tab_hbm.at[:, pl.ds(blk, 128)], buf_v.at[s], sem.at[s])

    for j in range(_NBUF):
        _issue(j, j)
    for j in range(_B_PER_W):
        s = j % _NBUF
        blk = (scal[j] // 128) * 128
        pltpu.make_async_copy(
            tab_hbm.at[:, pl.ds(blk, 128)], buf_v.at[s], sem.at[s]
        ).wait()
        lanes = jnp.full((16,), scal[j] % 128, jnp.int32)
        for g in range(EMBED // 16):
            rows = lax.iota(jnp.int32, 16) + 16 * g
            vals = plsc.load_gather(buf_v.at[s], [rows, lanes])
            rows_v[j, pl.ds(16 * g, 16)] = vals
        if j + _NBUF < _B_PER_W:
            _issue(j + _NBUF, s)
    pltpu.sync_copy(rows_v, out_hbm.at[pl.ds(base, _B_PER_W)])


def _sc_gather(idx, tab_t):
    mesh = plsc.VectorSubcoreMesh(core_axis_name="c", subcore_axis_name="s")
    return pl.kernel(
        _sc_gather_body,
        mesh=mesh,
        out_type=jax.ShapeDtypeStruct((BATCH, EMBED), jnp.float32),
        scratch_types=[
            pltpu.VMEM((_B_PER_W,), jnp.int32),
            pltpu.VMEM((_NBUF, EMBED, 128), jnp.float32),
            pltpu.VMEM((_B_PER_W, EMBED), jnp.float32),
            pltpu.SemaphoreType.DMA((_NBUF,)),
        ],
        compiler_params=pltpu.CompilerParams(needs_layout_passes=False),
    )(idx, tab_t)


_TV = 4096


def _mm_body(emb_ref, w_ref, b_ref, out_ref):
    w_aug = jnp.concatenate([w_ref[...], b_ref[...]], axis=0)
    emb_aug = jnp.concatenate(
        [emb_ref[...], jnp.ones((1, BATCH), jnp.float32)], axis=0
    )
    out_ref[...] = lax.dot_general(
        w_aug, emb_aug, (((0,), (0,)), ((), ())),
        preferred_element_type=jnp.float32,
    )


def _tc_project(emb_t, W, b2d):
    grid = (pl.cdiv(VOCAB, _TV),)
    return pl.pallas_call(
        _mm_body,
        grid=grid,
        in_specs=[
            pl.BlockSpec((EMBED, BATCH), lambda j: (0, 0)),
            pl.BlockSpec((EMBED, _TV), lambda j: (0, j)),
            pl.BlockSpec((1, _TV), lambda j: (0, j)),
        ],
        out_specs=pl.BlockSpec((_TV, BATCH), lambda j: (j, 0)),
        out_shape=jax.ShapeDtypeStruct((VOCAB, BATCH), jnp.float32),
        compiler_params=pltpu.CompilerParams(
            dimension_semantics=("arbitrary",),
            vmem_limit_bytes=100 * 1024 * 1024,
        ),
    )(emb_t, W, b2d)


def kernel(center_word, emb_table, W, b):
    idx = center_word.astype(jnp.int32)
    emb_t = _sc_gather(idx, emb_table.T).T
    logits_t = _tc_project(emb_t, W, b.reshape(1, VOCAB))
    return logits_t.T


# speedup vs baseline: 1.0148x; 1.0148x over previous
import functools

import jax
import jax.numpy as jnp
from jax import lax
from jax.experimental import pallas as pl
from jax.experimental.pallas import tpu as pltpu
from jax.experimental.pallas import tpu_sc as plsc

VOCAB = 100000
EMBED = 64
BATCH = 1024

_SC_INFO = plsc.get_sparse_core_info()
_NC = _SC_INFO.num_cores
_NS = _SC_INFO.num_subcores
_NW = _NC * _NS
_B_PER_W = BATCH // _NW


_NBUF = 8


def _sc_gather_body(idx_hbm, tab_hbm, out_hbm, idx_v, buf_v, rows_v, sem):
    wid = lax.axis_index("s") * _NC + lax.axis_index("c")
    base = wid * _B_PER_W
    pltpu.sync_copy(idx_hbm.at[pl.ds(base, _B_PER_W)], idx_v)
    lane_iota = lax.iota(jnp.int32, 16)
    scal = []
    for j in range(_B_PER_W):
        grp = idx_v[pl.ds((j // 16) * 16, 16)]
        scal.append(jnp.sum(jnp.where(lane_iota == (j % 16), grp, 0)))
    def _issue(j, s):
        blk = (scal[j] // 128) * 128
        pltpu.async_copy(tab_hbm.at[:, pl.ds(blk, 128)], buf_v.at[s], sem.at[s])

    for j in range(_NBUF):
        _issue(j, j)
    for j in range(_B_PER_W):
        s = j % _NBUF
        blk = (scal[j] // 128) * 128
        pltpu.make_async_copy(
            tab_hbm.at[:, pl.ds(blk, 128)], buf_v.at[s], sem.at[s]
        ).wait()
        lanes = jnp.full((16,), scal[j] % 128, jnp.int32)
        for g in range(EMBED // 16):
            rows = lax.iota(jnp.int32, 16) + 16 * g
            vals = plsc.load_gather(buf_v.at[s], [rows, lanes])
            rows_v[j, pl.ds(16 * g, 16)] = vals
        if j + _NBUF < _B_PER_W:
            _issue(j + _NBUF, s)
    pltpu.sync_copy(rows_v, out_hbm.at[pl.ds(base, _B_PER_W)])


def _sc_gather(idx, tab_t):
    mesh = plsc.VectorSubcoreMesh(core_axis_name="c", subcore_axis_name="s")
    return pl.kernel(
        _sc_gather_body,
        mesh=mesh,
        out_type=jax.ShapeDtypeStruct((BATCH, EMBED), jnp.float32),
        scratch_types=[
            pltpu.VMEM((_B_PER_W,), jnp.int32),
            pltpu.VMEM((_NBUF, EMBED, 128), jnp.float32),
            pltpu.VMEM((_B_PER_W, EMBED), jnp.float32),
            pltpu.SemaphoreType.DMA((_NBUF,)),
        ],
        compiler_params=pltpu.CompilerParams(needs_layout_passes=False),
    )(idx, tab_t)


_TV = 4096


def _mm_body(emb_ref, w_ref, b_ref, out_ref):
    w_aug = jnp.concatenate([w_ref[...], b_ref[...]], axis=0)
    emb_aug = jnp.concatenate(
        [emb_ref[...], jnp.ones((BATCH, 1), jnp.float32)], axis=1
    )
    out_ref[...] = lax.dot_general(
        w_aug, emb_aug, (((0,), (1,)), ((), ())),
        preferred_element_type=jnp.float32,
    )


def _tc_project(emb_t, W, b2d):
    grid = (pl.cdiv(VOCAB, _TV),)
    return pl.pallas_call(
        _mm_body,
        grid=grid,
        in_specs=[
            pl.BlockSpec((BATCH, EMBED), lambda j: (0, 0)),
            pl.BlockSpec((EMBED, _TV), lambda j: (0, j)),
            pl.BlockSpec((1, _TV), lambda j: (0, j)),
        ],
        out_specs=pl.BlockSpec((_TV, BATCH), lambda j: (j, 0)),
        out_shape=jax.ShapeDtypeStruct((VOCAB, BATCH), jnp.float32),
        compiler_params=pltpu.CompilerParams(
            dimension_semantics=("arbitrary",),
            vmem_limit_bytes=100 * 1024 * 1024,
        ),
    )(emb_t, W, b2d)


def kernel(center_word, emb_table, W, b):
    idx = center_word.astype(jnp.int32)
    emb = _sc_gather(idx, emb_table.T)
    logits_t = _tc_project(emb, W, b.reshape(1, VOCAB))
    return logits_t.T


# speedup vs baseline: 1.0161x; 1.0013x over previous
import functools

import jax
import jax.numpy as jnp
from jax import lax
from jax.experimental import pallas as pl
from jax.experimental.pallas import tpu as pltpu
from jax.experimental.pallas import tpu_sc as plsc

VOCAB = 100000
EMBED = 64
BATCH = 1024

_SC_INFO = plsc.get_sparse_core_info()
_NC = _SC_INFO.num_cores
_NS = _SC_INFO.num_subcores
_NW = _NC * _NS
_B_PER_W = BATCH // _NW


_NBUF = 8


def _sc_gather_body(idx_hbm, tab_hbm, out_hbm, idx_v, buf_v, rows_v, sem):
    wid = lax.axis_index("s") * _NC + lax.axis_index("c")
    base = wid * _B_PER_W
    pltpu.sync_copy(idx_hbm.at[pl.ds(base, _B_PER_W)], idx_v)
    lane_iota = lax.iota(jnp.int32, 16)
    scal = []
    for j in range(_B_PER_W):
        grp = idx_v[pl.ds((j // 16) * 16, 16)]
        scal.append(jnp.sum(jnp.where(lane_iota == (j % 16), grp, 0)))
    def _issue(j, s):
        blk = (scal[j] // 128) * 128
        pltpu.async_copy(tab_hbm.at[:, pl.ds(blk, 128)], buf_v.at[s], sem.at[s])

    for j in range(_NBUF):
        _issue(j, j)
    for j in range(_B_PER_W):
        s = j % _NBUF
        blk = (scal[j] // 128) * 128
        pltpu.make_async_copy(
            tab_hbm.at[:, pl.ds(blk, 128)], buf_v.at[s], sem.at[s]
        ).wait()
        lanes = jnp.full((16,), scal[j] % 128, jnp.int32)
        for g in range(EMBED // 16):
            rows = lax.iota(jnp.int32, 16) + 16 * g
            vals = plsc.load_gather(buf_v.at[s], [rows, lanes])
            rows_v[j, pl.ds(16 * g, 16)] = vals
        if j + _NBUF < _B_PER_W:
            _issue(j + _NBUF, s)
    pltpu.sync_copy(rows_v, out_hbm.at[pl.ds(base, _B_PER_W)])


def _sc_gather(idx, tab_t):
    mesh = plsc.VectorSubcoreMesh(core_axis_name="c", subcore_axis_name="s")
    return pl.kernel(
        _sc_gather_body,
        mesh=mesh,
        out_type=jax.ShapeDtypeStruct((BATCH, EMBED), jnp.float32),
        scratch_types=[
            pltpu.VMEM((_B_PER_W,), jnp.int32),
            pltpu.VMEM((_NBUF, EMBED, 128), jnp.float32),
            pltpu.VMEM((_B_PER_W, EMBED), jnp.float32),
            pltpu.SemaphoreType.DMA((_NBUF,)),
        ],
        compiler_params=pltpu.CompilerParams(needs_layout_passes=False),
    )(idx, tab_t)


_TV = 5120


def _mm_body(emb_ref, w_ref, b_ref, out_ref):
    w_aug = jnp.concatenate([w_ref[...], b_ref[...]], axis=0)
    emb_aug = jnp.concatenate(
        [emb_ref[...], jnp.ones((BATCH, 1), jnp.float32)], axis=1
    )
    out_ref[...] = lax.dot_general(
        w_aug, emb_aug, (((0,), (1,)), ((), ())),
        preferred_element_type=jnp.float32,
    )


def _tc_project(emb_t, W, b2d):
    grid = (pl.cdiv(VOCAB, _TV),)
    return pl.pallas_call(
        _mm_body,
        grid=grid,
        in_specs=[
            pl.BlockSpec((BATCH, EMBED), lambda j: (0, 0)),
            pl.BlockSpec((EMBED, _TV), lambda j: (0, j)),
            pl.BlockSpec((1, _TV), lambda j: (0, j)),
        ],
        out_specs=pl.BlockSpec((_TV, BATCH), lambda j: (j, 0)),
        out_shape=jax.ShapeDtypeStruct((VOCAB, BATCH), jnp.float32),
        compiler_params=pltpu.CompilerParams(
            dimension_semantics=("arbitrary",),
            vmem_limit_bytes=100 * 1024 * 1024,
        ),
    )(emb_t, W, b2d)


def kernel(center_word, emb_table, W, b):
    idx = center_word.astype(jnp.int32)
    emb = _sc_gather(idx, emb_table.T)
    logits_t = _tc_project(emb, W, b.reshape(1, VOCAB))
    return logits_t.T
